# baseline (speedup 1.0000x reference)
import functools

import jax
import jax.numpy as jnp
from jax import lax
from jax.experimental import pallas as pl
from jax.experimental.pallas import tpu as pltpu
from jax.experimental.pallas import tpu_sc as plsc

DIM = 64
GAMMA = 12.0
L = 16
NCHUNK = DIM // L
NC, NS = 2, 16
NW = NC * NS
STG = 133


@functools.lru_cache(maxsize=None)
def _make_transpose_kernel(NENT):
    EW = 256
    NB = NENT // EW
    rem_p = (NENT - NB * EW) // 2
    NPB = -(-NB // NW)
    NPAIR = -(-NPB // 2)
    SST = EW + 5

    mesh = plsc.VectorSubcoreMesh(
        core_axis_name="c", subcore_axis_name="s",
        num_cores=NC, num_subcores=NS)

    @functools.partial(
        pl.kernel,
        out_type=jax.ShapeDtypeStruct((NENT // 2, 128), jnp.float32),
        mesh=mesh,
        compiler_params=pltpu.CompilerParams(
            needs_layout_passes=False, use_tc_tiling_on_sc=True),
        scratch_types=[
            pltpu.VMEM((DIM, SST), jnp.float32),
            pltpu.VMEM((DIM, SST), jnp.float32),
            pltpu.VMEM((EW // 2, 128), jnp.float32),
            pltpu.VMEM((EW // 2, 128), jnp.float32),
            pltpu.VMEM((max(rem_p, 1), 128), jnp.float32),
            pltpu.SemaphoreType.DMA,
            pltpu.SemaphoreType.DMA,
            pltpu.SemaphoreType.DMA,
            pltpu.SemaphoreType.DMA,
            pltpu.SemaphoreType.DMA,
        ],
    )
    def k(ent_t_hbm, etail_hbm, out_hbm,
          in0, in1, tout0, tout1, rbuf, rs0, rs1, ws0, ws1, rsem):
        wid = lax.axis_index("s") * NC + lax.axis_index("c")
        lane_iota = lax.iota(jnp.int32, L)

        def start_read(q, buf, sem):
            @pl.when(q < NB)
            def _():
                pltpu.async_copy(
                    ent_t_hbm.at[:, pl.ds(q * EW, EW)],
                    buf.at[:, pl.ds(0, EW)], sem)

        start_read(wid, in0, rs0)
        start_read(wid + NW, in1, rs1)

        def step(it, par, bin_, bout, rsem_, wsem):
            q = wid + NW * (2 * it + par)

            @pl.when(q < NB)
            def _():
                pltpu.make_async_copy(
                    ent_t_hbm.at[:, pl.ds(0, EW)],
                    bin_.at[:, pl.ds(0, EW)], rsem_).wait()

                def row_body(r4, carry):
                    for u in range(4):
                        r = r4 * 4 + u
                        rvec = jnp.full((L,), r, jnp.int32)
                        half = (r & 1) * DIM
                        p = r >> 1
                        for c in range(NCHUNK):
                            vals = plsc.load_gather(
                                bin_, [c * L + lane_iota, rvec])
                            bout[p, pl.ds(half + c * L, L)] = vals
                    return carry
                lax.fori_loop(0, EW // 4, row_body, 0)

                @pl.when(it > 0)
                def _():
                    pltpu.make_async_copy(
                        bout, out_hbm.at[pl.ds(0, EW // 2)], wsem).wait()
                pltpu.async_copy(
                    bout, out_hbm.at[pl.ds(q * (EW // 2), EW // 2)], wsem)
                start_read(q + 2 * NW, bin_, rsem_)

        def body(it, carry):
            step(it, 0, in0, tout0, rs0, ws0)
            step(it, 1, in1, tout1, rs1, ws1)
            return carry
        lax.fori_loop(0, NPAIR, body, 0)

        pltpu.make_async_copy(
            tout0, out_hbm.at[pl.ds(0, EW // 2)], ws0).wait()
        pltpu.make_async_copy(
            tout1, out_hbm.at[pl.ds(0, EW // 2)], ws1).wait()

        if rem_p:
            @pl.when(wid == NW - 1)
            def _():
                pltpu.async_copy(etail_hbm, rbuf, rsem).wait()
                pltpu.sync_copy(
                    rbuf, out_hbm.at[pl.ds(NB * (EW // 2), rem_p)])

    return k


@functools.lru_cache(maxsize=None)
def _make_score_kernel(B, NEG, NENT2):
    rows_pw = B // NW
    TPT = 128
    halves = NEG // TPT
    tasks_pw = rows_pw * halves

    mesh = plsc.VectorSubcoreMesh(
        core_axis_name="c", subcore_axis_name="s",
        num_cores=NC, num_subcores=NS)

    @functools.partial(
        pl.kernel,
        out_type=jax.ShapeDtypeStruct((B * halves, TPT), jnp.float32),
        mesh=mesh,
        compiler_params=pltpu.CompilerParams(
            needs_layout_passes=False, use_tc_tiling_on_sc=True),
        scratch_types=[
            pltpu.VMEM((rows_pw,), jnp.int32),
            pltpu.VMEM((rows_pw,), jnp.int32),
            pltpu.VMEM((rows_pw,), jnp.int32),
            pltpu.VMEM((rows_pw,), jnp.int32),
            pltpu.VMEM((rows_pw, 2 * DIM), jnp.float32),
            pltpu.VMEM((rows_pw, 2 * DIM), jnp.float32),
            pltpu.VMEM((rows_pw, DIM), jnp.float32),
            pltpu.VMEM((tasks_pw, TPT), jnp.int32),
            pltpu.VMEM((TPT,), jnp.int32),
            pltpu.VMEM((TPT,), jnp.int32),
            pltpu.VMEM((TPT, 2 * DIM), jnp.float32),
            pltpu.VMEM((TPT, 2 * DIM), jnp.float32),
            pltpu.VMEM((TPT,), jnp.float32),
            pltpu.VMEM((TPT,), jnp.float32),
            pltpu.SemaphoreType.DMA,
            pltpu.SemaphoreType.DMA,
            pltpu.SemaphoreType.DMA,
            pltpu.SemaphoreType.DMA,
            pltpu.SemaphoreType.DMA,
        ],
    )
    def k(hidx_hbm, ridx_hbm, tidx_hbm, ent_hbm, rel_hbm, out_hbm,
          hidx_v, ridx_v, hg_v, rg_v, head_v, relv_v, hr_v, tidx_v,
          gidx0, gidx1, tails0, tails1, scores0, scores1,
          gsem0, gsem1, osem0, osem1, psem):
        wid = lax.axis_index("s") * NC + lax.axis_index("c")
        base_row = wid * rows_pw
        base_task = wid * tasks_pw
        lane_iota = lax.iota(jnp.int32, L)

        pltpu.sync_copy(hidx_hbm.at[pl.ds(base_row, rows_pw)], hidx_v)
        pltpu.sync_copy(ridx_hbm.at[pl.ds(base_row, rows_pw)], ridx_v)
        for c in range(rows_pw // L):
            sl = pl.ds(c * L, L)
            hg_v[sl] = hidx_v[sl] >> 1
            rg_v[sl] = ridx_v[sl] >> 1
        cp_t = pltpu.async_copy(
            tidx_hbm.at[pl.ds(base_task, tasks_pw)], tidx_v, psem)
        cp_h = pltpu.async_copy(ent_hbm.at[hg_v], head_v, psem)
        cp_r = pltpu.async_copy(rel_hbm.at[rg_v], relv_v, psem)
        cp_t.wait()
        cp_h.wait()
        cp_r.wait()

        for rg in range(rows_pw // L):
            rows = rg * L + lane_iota
            hoffs = (hidx_v[pl.ds(rg * L, L)] & 1) * DIM
            roffs = (ridx_v[pl.ds(rg * L, L)] & 1) * DIM
            for d in range(DIM):
                hv = plsc.load_gather(head_v, [rows, hoffs + d])
                rv = plsc.load_gather(relv_v, [rows, roffs + d])
                plsc.store_scatter(
                    hr_v, [rows, jnp.full((L,), d, jnp.int32)], hv + rv)

        def fill_gidx(t, gidx):
            for c in range(TPT // L):
                sl = pl.ds(c * L, L)
                gidx[sl] = tidx_v[t, sl] >> 1

        fill_gidx(0, gidx0)
        fill_gidx(1, gidx1)
        pltpu.async_copy(ent_hbm.at[gidx0], tails0, gsem0)
        pltpu.async_copy(ent_hbm.at[gidx1], tails1, gsem1)

        def run_task(i, par, gidx, tails, scores, gsem, osem):
            t = halves * i + par
            pltpu.make_async_copy(ent_hbm.at[gidx], tails, gsem).wait()
            @pl.when(i > 0)
            def _():
                pltpu.make_async_copy(
                    scores, out_hbm.at[base_task], osem).wait()

            rowvec_i = jnp.full((L,), i, jnp.int32)

            def group_body(g, carry):
                sbase = g * L
                rows = sbase + lane_iota
                cols0 = (tidx_v[t, pl.ds(sbase, L)] & 1) * DIM
                acc0 = jnp.zeros((L,), jnp.float32)
                acc1 = jnp.zeros((L,), jnp.float32)
                for step in range(DIM):
                    dvec = (lane_iota + step) & (DIM - 1)
                    vals = plsc.load_gather(tails, [rows, cols0 + dvec])
                    hrv = plsc.load_gather(hr_v, [rowvec_i, dvec])
                    if step % 2 == 0:
                        acc0 = acc0 + jnp.abs(hrv - vals)
                    else:
                        acc1 = acc1 + jnp.abs(hrv - vals)
                scores[pl.ds(sbase, L)] = GAMMA - (acc0 + acc1)
                return carry
            lax.fori_loop(0, TPT // L, group_body, 0)

            pltpu.async_copy(scores, out_hbm.at[base_task + t], osem)
            @pl.when(i < rows_pw - 1)
            def _():
                fill_gidx(t + halves, gidx)
                pltpu.async_copy(ent_hbm.at[gidx], tails, gsem)

        def loop_body(i, carry):
            run_task(i, 0, gidx0, tails0, scores0, gsem0, osem0)
            run_task(i, 1, gidx1, tails1, scores1, gsem1, osem1)
            return carry
        lax.fori_loop(0, rows_pw, loop_body, 0)

        pltpu.make_async_copy(scores0, out_hbm.at[base_task], osem0).wait()
        pltpu.make_async_copy(scores1, out_hbm.at[base_task], osem1).wait()

    return k


def kernel(head_part, tail_part, relative_dist, entity_embedding,
           relation_embedding, relation_head, relation_tail):
    B, NEG = tail_part.shape
    NENT, D = entity_embedding.shape
    NRELT = relation_embedding.shape[0]

    ent_t = entity_embedding.T
    nb = NENT // 128
    etail = entity_embedding[nb * 128:].reshape(-1, 128)
    ent2 = _make_transpose_kernel(NENT)(ent_t, etail)

    h_idx = head_part[:, 0].astype(jnp.int32)
    r_idx = head_part[:, 1].astype(jnp.int32)
    tidx = tail_part.astype(jnp.int32).reshape(B * (NEG // 128), 128)
    rel2 = relation_embedding.reshape(NRELT // 2, 2 * D)
    k = _make_score_kernel(B, NEG, NENT // 2)
    out = k(h_idx, r_idx, tidx, ent2, rel2)
    return out.reshape(B, NEG)


# speedup vs baseline: 2.0216x; 2.0216x over previous
import functools

import jax
import jax.numpy as jnp
from jax import lax
from jax.experimental import pallas as pl
from jax.experimental.pallas import tpu as pltpu
from jax.experimental.pallas import tpu_sc as plsc

DIM = 64
GAMMA = 12.0
L = 16
NCHUNK = DIM // L
NC, NS = 2, 16
NW = NC * NS
STG = 133


@functools.lru_cache(maxsize=None)
def _make_tc_transpose(NENT):
    BLK = 512
    nblk = -(-(NENT // 2) // BLK)
    F = nblk * BLK

    def body(x0_ref, x1_ref, o_ref):
        o_ref[:, 0:DIM] = x0_ref[...].T
        o_ref[:, DIM:2 * DIM] = x1_ref[...].T

    call = pl.pallas_call(
        body,
        out_shape=jax.ShapeDtypeStruct((F, 2 * DIM), jnp.float32),
        grid=(nblk,),
        in_specs=[
            pl.BlockSpec((DIM, BLK), lambda i: (0, i)),
            pl.BlockSpec((DIM, BLK), lambda i: (0, i + nblk)),
        ],
        out_specs=pl.BlockSpec((BLK, 2 * DIM), lambda i: (i, 0)),
    )
    return call, F


@functools.lru_cache(maxsize=None)
def _make_score_kernel(B, NEG, F):
    rows_pw = B // NW
    TPT = 128
    halves = NEG // TPT
    tasks_pw = rows_pw * halves

    mesh = plsc.VectorSubcoreMesh(
        core_axis_name="c", subcore_axis_name="s",
        num_cores=NC, num_subcores=NS)

    @functools.partial(
        pl.kernel,
        out_type=jax.ShapeDtypeStruct((B * halves, TPT), jnp.float32),
        mesh=mesh,
        compiler_params=pltpu.CompilerParams(
            needs_layout_passes=False, use_tc_tiling_on_sc=True),
        scratch_types=[
            pltpu.VMEM((rows_pw,), jnp.int32),
            pltpu.VMEM((rows_pw,), jnp.int32),
            pltpu.VMEM((rows_pw,), jnp.int32),
            pltpu.VMEM((rows_pw,), jnp.int32),
            pltpu.VMEM((rows_pw, 2 * DIM), jnp.float32),
            pltpu.VMEM((rows_pw, 2 * DIM), jnp.float32),
            pltpu.VMEM((rows_pw, DIM), jnp.float32),
            pltpu.VMEM((tasks_pw, TPT), jnp.int32),
            pltpu.VMEM((TPT,), jnp.int32),
            pltpu.VMEM((TPT,), jnp.int32),
            pltpu.VMEM((TPT, 2 * DIM), jnp.float32),
            pltpu.VMEM((TPT, 2 * DIM), jnp.float32),
            pltpu.VMEM((TPT,), jnp.float32),
            pltpu.VMEM((TPT,), jnp.float32),
            pltpu.SemaphoreType.DMA,
            pltpu.SemaphoreType.DMA,
            pltpu.SemaphoreType.DMA,
            pltpu.SemaphoreType.DMA,
            pltpu.SemaphoreType.DMA,
        ],
    )
    def k(hidx_hbm, ridx_hbm, tidx_hbm, ent_hbm, rel_hbm, out_hbm,
          hidx_v, ridx_v, hg_v, rg_v, head_v, relv_v, hr_v, tidx_v,
          gidx0, gidx1, tails0, tails1, scores0, scores1,
          gsem0, gsem1, osem0, osem1, psem):
        wid = lax.axis_index("s") * NC + lax.axis_index("c")
        base_row = wid * rows_pw
        base_task = wid * tasks_pw
        lane_iota = lax.iota(jnp.int32, L)

        pltpu.sync_copy(hidx_hbm.at[pl.ds(base_row, rows_pw)], hidx_v)
        pltpu.sync_copy(ridx_hbm.at[pl.ds(base_row, rows_pw)], ridx_v)
        for c in range(rows_pw // L):
            sl = pl.ds(c * L, L)
            hch = hidx_v[sl]
            hg_v[sl] = jnp.where(hch >= F, hch - F, hch)
            rg_v[sl] = ridx_v[sl] >> 1
        cp_t = pltpu.async_copy(
            tidx_hbm.at[pl.ds(base_task, tasks_pw)], tidx_v, psem)
        cp_h = pltpu.async_copy(ent_hbm.at[hg_v], head_v, psem)
        cp_r = pltpu.async_copy(rel_hbm.at[rg_v], relv_v, psem)
        cp_t.wait()
        cp_h.wait()
        cp_r.wait()

        for rg in range(rows_pw // L):
            rows = rg * L + lane_iota
            hoffs = jnp.where(hidx_v[pl.ds(rg * L, L)] >= F, DIM, 0)
            roffs = (ridx_v[pl.ds(rg * L, L)] & 1) * DIM
            for d in range(DIM):
                hv = plsc.load_gather(head_v, [rows, hoffs + d])
                rv = plsc.load_gather(relv_v, [rows, roffs + d])
                plsc.store_scatter(
                    hr_v, [rows, jnp.full((L,), d, jnp.int32)], hv + rv)

        def fill_gidx(t, gidx):
            for c in range(TPT // L):
                sl = pl.ds(c * L, L)
                tch = tidx_v[t, sl]
                gidx[sl] = jnp.where(tch >= F, tch - F, tch)

        fill_gidx(0, gidx0)
        fill_gidx(1, gidx1)
        pltpu.async_copy(ent_hbm.at[gidx0], tails0, gsem0)
        pltpu.async_copy(ent_hbm.at[gidx1], tails1, gsem1)

        def run_task(i, par, gidx, tails, scores, gsem, osem):
            t = halves * i + par
            pltpu.make_async_copy(ent_hbm.at[gidx], tails, gsem).wait()
            @pl.when(i > 0)
            def _():
                pltpu.make_async_copy(
                    scores, out_hbm.at[base_task], osem).wait()

            rowvec_i = jnp.full((L,), i, jnp.int32)

            def group_body(g, carry):
                sbase = g * L
                rows = sbase + lane_iota
                cols0 = jnp.where(tidx_v[t, pl.ds(sbase, L)] >= F, DIM, 0)
                acc0 = jnp.zeros((L,), jnp.float32)
                acc1 = jnp.zeros((L,), jnp.float32)
                for step in range(DIM):
                    dvec = (lane_iota + step) & (DIM - 1)
                    vals = plsc.load_gather(tails, [rows, cols0 + dvec])
                    hrv = plsc.load_gather(hr_v, [rowvec_i, dvec])
                    if step % 2 == 0:
                        acc0 = acc0 + jnp.abs(hrv - vals)
                    else:
                        acc1 = acc1 + jnp.abs(hrv - vals)
                scores[pl.ds(sbase, L)] = GAMMA - (acc0 + acc1)
                return carry
            lax.fori_loop(0, TPT // L, group_body, 0)

            pltpu.async_copy(scores, out_hbm.at[base_task + t], osem)
            @pl.when(i < rows_pw - 1)
            def _():
                fill_gidx(t + halves, gidx)
                pltpu.async_copy(ent_hbm.at[gidx], tails, gsem)

        def loop_body(i, carry):
            run_task(i, 0, gidx0, tails0, scores0, gsem0, osem0)
            run_task(i, 1, gidx1, tails1, scores1, gsem1, osem1)
            return carry
        lax.fori_loop(0, rows_pw, loop_body, 0)

        pltpu.make_async_copy(scores0, out_hbm.at[base_task], osem0).wait()
        pltpu.make_async_copy(scores1, out_hbm.at[base_task], osem1).wait()

    return k


def kernel(head_part, tail_part, relative_dist, entity_embedding,
           relation_embedding, relation_head, relation_tail):
    B, NEG = tail_part.shape
    NENT, D = entity_embedding.shape
    NRELT = relation_embedding.shape[0]

    tc_call, F = _make_tc_transpose(NENT)
    ent_t = entity_embedding.T
    ent2 = tc_call(ent_t, ent_t)

    h_idx = head_part[:, 0].astype(jnp.int32)
    r_idx = head_part[:, 1].astype(jnp.int32)
    tidx = tail_part.astype(jnp.int32).reshape(B * (NEG // 128), 128)
    rel2 = relation_embedding.reshape(NRELT // 2, 2 * D)
    k = _make_score_kernel(B, NEG, F)
    out = k(h_idx, r_idx, tidx, ent2, rel2)
    return out.reshape(B, NEG)
